# baseline (speedup 1.0000x reference)
import functools

import jax
import jax.numpy as jnp
from jax import lax
from jax.experimental import pallas as pl
from jax.experimental.pallas import tpu as pltpu
from jax.experimental.pallas import tpu_sc as plsc

N = 10000
E = 320000
H = 128
HH = H // 2
NORM = 100.0

NW = 32
EW = E // NW

CW = 80
RING = 5
NSL = 5
ES = E // NSL
NCH = ES // (NW * CW)

SCS = 2000
SNCH = ES // (NW * SCS)
NP = 10240


def _pack_bf16(lo_f32, hi_f32):
    ulo = lax.bitcast_convert_type(lo_f32, jnp.int32)
    uhi = lax.bitcast_convert_type(hi_f32, jnp.int32)
    lo = lax.shift_right_logical(ulo + 0x8000, 16)
    hi = (uhi + 0x8000) & jnp.int32(-65536)
    return lo | hi


def _unpack_bf16(packed_i32):
    lo = lax.bitcast_convert_type(lax.shift_left(packed_i32, 16), jnp.float32)
    hi = lax.bitcast_convert_type(packed_i32 & jnp.int32(-65536), jnp.float32)
    return lo, hi


def _pre_body(h_ref, w1ae_ref, w1ao_ref, w1be_ref, w1bo_ref, t_ref):
    hblk = h_ref[...]
    pae = jnp.dot(hblk, w1ae_ref[...], preferred_element_type=jnp.float32)
    pao = jnp.dot(hblk, w1ao_ref[...], preferred_element_type=jnp.float32)
    pbe = jnp.dot(hblk, w1be_ref[...], preferred_element_type=jnp.float32)
    pbo = jnp.dot(hblk, w1bo_ref[...], preferred_element_type=jnp.float32)
    t_ref[...] = jnp.concatenate(
        [_pack_bf16(pae, pao), _pack_bf16(pbe, pbo)], axis=1)


def _precompute(h, w1ae, w1ao, w1be, w1bo):
    bn = 2000
    grid = (N // bn,)
    wspec = pl.BlockSpec((H, HH), lambda i: (0, 0))
    return pl.pallas_call(
        _pre_body,
        grid=grid,
        in_specs=[pl.BlockSpec((bn, H), lambda i: (i, 0))] + [wspec] * 4,
        out_specs=pl.BlockSpec((bn, H), lambda i: (i, 0)),
        out_shape=jax.ShapeDtypeStruct((N, H), jnp.int32),
    )(h, w1ae, w1ao, w1be, w1bo)


def _gather_body(t_hbm, row_hbm, col_hbm, ga_hbm, gb_hbm,
                 idxa, idxb, rawa, rawb, bufa, bufb, *sems):
    gsa = sems[0:RING]
    gsb = sems[RING:2 * RING]
    wsa = sems[2 * RING:3 * RING]
    wsb = sems[3 * RING:4 * RING]
    c = lax.axis_index("c")
    s = lax.axis_index("s")
    wid = s * 2 + c
    cbase = wid * NCH
    iota = lax.iota(jnp.int32, 16)
    EWH = NCH * CW // 2

    pltpu.sync_copy(row_hbm.at[pl.ds(wid * EWH, EWH)], rawa.at[pl.ds(0, EWH)])
    pltpu.sync_copy(row_hbm.at[pl.ds(ES // 2 + wid * EWH, EWH)],
                    rawa.at[pl.ds(EWH, EWH)])
    pltpu.sync_copy(col_hbm.at[pl.ds(wid * EWH, EWH)], rawb.at[pl.ds(0, EWH)])
    pltpu.sync_copy(col_hbm.at[pl.ds(ES // 2 + wid * EWH, EWH)],
                    rawb.at[pl.ds(EWH, EWH)])

    def ilv(g, carry):
        l16 = g * 16 + iota
        msk = l16 < EWH
        for raw, idx in ((rawa, idxa), (rawb, idxb)):
            lo16 = raw[pl.ds(g * 16, 16)]
            hi16 = raw[pl.ds(EWH + g * 16, 16)]
            plsc.store_scatter(idx, [2 * l16], lo16, mask=msk)
            plsc.store_scatter(idx, [2 * l16 + 1], hi16, mask=msk)
        return carry

    lax.fori_loop(0, (EWH + 15) // 16, ilv, 0)

    def do_fire(j, b):
        off = j * CW
        pltpu.async_copy(t_hbm.at[idxa.at[pl.ds(off, CW)]], bufa.at[b], gsa[b])
        pltpu.async_copy(t_hbm.at[idxb.at[pl.ds(off, CW)]], bufb.at[b], gsb[b])

    def do_writeout(j, b):
        off = j * CW
        pltpu.make_async_copy(t_hbm.at[idxa.at[pl.ds(off, CW)]],
                              bufa.at[b], gsa[b]).wait()
        pltpu.make_async_copy(t_hbm.at[idxb.at[pl.ds(off, CW)]],
                              bufb.at[b], gsb[b]).wait()
        off = (cbase + j) * CW
        pltpu.async_copy(bufa.at[b], ga_hbm.at[pl.ds(off, CW)], wsa[b])
        pltpu.async_copy(bufb.at[b], gb_hbm.at[pl.ds(off, CW)], wsb[b])

    def drain_writeout(b):
        pltpu.make_async_copy(bufa.at[b], ga_hbm.at[pl.ds(0, CW)], wsa[b]).wait()
        pltpu.make_async_copy(bufb.at[b], gb_hbm.at[pl.ds(0, CW)], wsb[b]).wait()

    def outer(go, carry):
        for b in range(RING):
            j = go * RING + b
            @pl.when(j >= 2)
            def _():
                do_writeout(j - 2, (b - 2) % RING)

            @pl.when(go >= 1)
            def _():
                drain_writeout(b)

            do_fire(j, b)
        return carry

    lax.fori_loop(0, NCH // RING, outer, 0)
    for j in (NCH - 2, NCH - 1):
        do_writeout(j, j % RING)
    for b in range(RING):
        drain_writeout(b)


def _gather(t2, row2, col2):
    mesh = plsc.VectorSubcoreMesh(core_axis_name="c", subcore_axis_name="s")
    k = pl.kernel(
        _gather_body,
        out_type=(
            jax.ShapeDtypeStruct((ES, HH), jnp.int32),
            jax.ShapeDtypeStruct((ES, HH), jnp.int32),
        ),
        mesh=mesh,
        scratch_types=[
            pltpu.VMEM((NCH * CW,), jnp.int32),
            pltpu.VMEM((NCH * CW,), jnp.int32),
            pltpu.VMEM((NCH * CW + 16,), jnp.int32),
            pltpu.VMEM((NCH * CW + 16,), jnp.int32),
            pltpu.VMEM((RING, CW, HH), jnp.int32),
            pltpu.VMEM((RING, CW, HH), jnp.int32),
        ] + [pltpu.SemaphoreType.DMA] * (4 * RING),
        compiler_params=pltpu.CompilerParams(use_tc_tiling_on_sc=False,
                                             needs_layout_passes=False),
    )
    return k(t2, row2, col2)


BK = 12800
BKH = BK // 2
BKR = BKH // H


def _mlp_body(ga_ref, gb_ref, eaL_ref, eaR_ref, w1ce_ref, w1co_ref,
              b1e_ref, b1o_ref, w2te_ref, w2to_ref, b2_ref, w3_ref,
              phiL_ref, phiR_ref):
    ale, alo = _unpack_bf16(ga_ref[...])
    ble, blo = _unpack_bf16(gb_ref[...])

    def half(lane0, ea_ref, phi_ref):
        sl = (slice(None), slice(lane0, lane0 + HH))
        xe = ale[sl] + ble[sl] + b1e_ref[...]
        xo = alo[sl] + blo[sl] + b1o_ref[...]
        ea3 = ea_ref[...][0][:, :, None]
        xe = xe + jnp.reshape(
            ea3 * jnp.reshape(w1ce_ref[...], (1, 1, HH)), (BKH, HH))
        xo = xo + jnp.reshape(
            ea3 * jnp.reshape(w1co_ref[...], (1, 1, HH)), (BKH, HH))
        xe = xe * (1.0 / (1.0 + jnp.exp(-xe)))
        xo = xo * (1.0 / (1.0 + jnp.exp(-xo)))
        x = (jnp.dot(xe.astype(jnp.bfloat16), w2te_ref[...],
                     preferred_element_type=jnp.float32)
             + jnp.dot(xo.astype(jnp.bfloat16), w2to_ref[...],
                       preferred_element_type=jnp.float32)
             + b2_ref[...])
        x = x * (1.0 / (1.0 + jnp.exp(-x)))
        phi = jnp.sum(jnp.reshape(x * w3_ref[...], (BKR, H, H)), axis=2)
        phi_ref[...] = jnp.reshape(phi, (1, BKR, H))

    half(0, eaL_ref, phiL_ref)
    half(HH, eaR_ref, phiR_ref)


def _edge_mlp(ga2, gb2, eaL, eaR, w1ce, w1co, b1e, b1o, w2te, w2to, b2r, w3r):
    grid = (ES // BK,)
    hspec = pl.BlockSpec((1, HH), lambda i: (0, 0))
    wspec = pl.BlockSpec((1, H), lambda i: (0, 0))
    easpec = pl.BlockSpec((1, BKR, H), lambda i: (i, 0, 0))
    pspec = pl.BlockSpec((1, BKR, H), lambda i: (i, 0, 0))
    pshape = jax.ShapeDtypeStruct((ES // BK, BKR, H), jnp.float32)
    return pl.pallas_call(
        _mlp_body,
        grid=grid,
        in_specs=[
            pl.BlockSpec((BKH, H), lambda i: (i, 0)),
            pl.BlockSpec((BKH, H), lambda i: (i, 0)),
            easpec,
            easpec,
            hspec,
            hspec,
            hspec,
            hspec,
            pl.BlockSpec((HH, H), lambda i: (0, 0)),
            pl.BlockSpec((HH, H), lambda i: (0, 0)),
            wspec,
            wspec,
        ],
        out_specs=[pspec, pspec],
        out_shape=[pshape, pshape],
    )(ga2, gb2, eaL.reshape(ES // BK, BKR, H), eaR.reshape(ES // BK, BKR, H),
      w1ce, w1co, b1e, b1o, w2te, w2to, b2r, w3r)


def _scatter_body(row_hbm, phi_hbm, cd0_hbm, cd1_hbm, cd2_hbm, part_hbm,
                  idx_v, phi_v, cd0_v, cd1_v, cd2_v, acc_v):
    c = lax.axis_index("c")
    s = lax.axis_index("s")
    wid = s * 2 + c
    base = wid * (ES // NW)
    iota = lax.iota(jnp.int32, 16)
    zeros16 = jnp.zeros((16,), jnp.float32)

    def zero(i, carry):
        for k in range(10):
            acc_v[pl.ds(i * 160 + k * 16, 16)] = zeros16
        return carry

    lax.fori_loop(0, 3 * NP // 160, zero, 0)

    def chunk(j, carry):
        off = base + j * SCS
        pltpu.sync_copy(row_hbm.at[pl.ds(off, SCS)], idx_v)
        pltpu.sync_copy(phi_hbm.at[pl.ds(off, SCS)], phi_v)
        pltpu.sync_copy(cd0_hbm.at[pl.ds(off, SCS)], cd0_v)
        pltpu.sync_copy(cd1_hbm.at[pl.ds(off, SCS)], cd1_v)
        pltpu.sync_copy(cd2_hbm.at[pl.ds(off, SCS)], cd2_v)

        def grp(g, carry2):
            p16 = phi_v[pl.ds(g * 16, 16)]
            e16 = idx_v[pl.ds(g * 16, 16)]
            for comp, cdv in enumerate((cd0_v, cd1_v, cd2_v)):
                cdc = cdv[pl.ds(g * 16, 16)]
                plsc.addupdate_scatter(acc_v, [e16 + comp * NP], p16 * cdc)
            return carry2

        lax.fori_loop(0, SCS // 16, grp, 0)
        return carry

    lax.fori_loop(0, SNCH, chunk, 0)
    pltpu.sync_copy(acc_v, part_hbm.at[wid])


def _scatter(row, phi, cd0, cd1, cd2):
    mesh = plsc.VectorSubcoreMesh(core_axis_name="c", subcore_axis_name="s")
    k = pl.kernel(
        _scatter_body,
        out_type=jax.ShapeDtypeStruct((NW, 3 * NP), jnp.float32),
        mesh=mesh,
        scratch_types=[
            pltpu.VMEM((SCS,), jnp.int32),
            pltpu.VMEM((SCS,), jnp.float32),
            pltpu.VMEM((SCS,), jnp.float32),
            pltpu.VMEM((SCS,), jnp.float32),
            pltpu.VMEM((SCS,), jnp.float32),
            pltpu.VMEM((3 * NP,), jnp.float32),
        ],
        compiler_params=pltpu.CompilerParams(needs_layout_passes=False),
    )
    return k(row, phi, cd0, cd1, cd2)


def _comb_body(*refs):
    part_refs = refs[:NSL]
    coordt_ref, out_ref = refs[NSL], refs[NSL + 1]
    p = part_refs[0][...]
    for pr in part_refs[1:]:
        p = p + pr[...]
    planes = [jnp.sum(p[:, comp * NP:(comp + 1) * NP], axis=0)[:N]
              for comp in range(3)]
    out_ref[...] = coordt_ref[...] + jnp.stack(planes, axis=0) * (1.0 / NORM)


def _combine(parts_list, coordt):
    return pl.pallas_call(
        _comb_body,
        grid=(1,),
        in_specs=[pl.BlockSpec((NW, 3 * NP), lambda i: (0, 0))] * NSL + [
            pl.BlockSpec((3, N), lambda i: (0, 0)),
        ],
        out_specs=pl.BlockSpec((3, N), lambda i: (0, 0)),
        out_shape=jax.ShapeDtypeStruct((3, N), jnp.float32),
    )(*parts_list, coordt)


def kernel(h, coord, edge_index, coord_diff, edge_attr, W1, b1, W2, b2, W3):
    row = edge_index[0]
    col = edge_index[1]
    w1at = W1[:, :H].T
    w1bt = W1[:, H:2 * H].T
    w1c = W1[:, 2 * H:].T
    w2t = W2.T
    w1ce, w1co = w1c[:, :HH], w1c[:, HH:]
    b1e = b1[:HH].reshape(1, HH)
    b1o = b1[HH:].reshape(1, HH)
    w2te = w2t[:HH, :].astype(jnp.bfloat16)
    w2to = w2t[HH:, :].astype(jnp.bfloat16)
    b2r = b2.reshape(1, H)
    w3r = W3
    t2 = _precompute(h, w1at[:, :HH], w1at[:, HH:],
                     w1bt[:, :HH], w1bt[:, HH:]).reshape(2 * N, HH)
    row2 = row * 2
    col2 = col * 2 + 1
    eat = edge_attr.T
    cdt = coord_diff.T
    parts_list = []
    for s in range(NSL):
        ga, gb = _gather(t2, row2[s * ES:(s + 1) * ES],
                         col2[s * ES:(s + 1) * ES])
        phiL, phiR = _edge_mlp(ga.reshape(ES // 2, H), gb.reshape(ES // 2, H),
                               eat[:, s * ES:s * ES + ES // 2],
                               eat[:, s * ES + ES // 2:(s + 1) * ES],
                               w1ce, w1co, b1e, b1o, w2te, w2to, b2r, w3r)
        phi_s = jnp.concatenate(
            [phiL.reshape(ES // 2), phiR.reshape(ES // 2)], axis=0)
        parts_list.append(_scatter(
            row[s * ES:(s + 1) * ES], phi_s,
            cdt[0, s * ES:(s + 1) * ES],
            cdt[1, s * ES:(s + 1) * ES],
            cdt[2, s * ES:(s + 1) * ES]))
    return _combine(parts_list, coord.T).T


# speedup vs baseline: 1.0451x; 1.0451x over previous
import functools

import jax
import jax.numpy as jnp
from jax import lax
from jax.experimental import pallas as pl
from jax.experimental.pallas import tpu as pltpu
from jax.experimental.pallas import tpu_sc as plsc

N = 10000
E = 320000
H = 128
HH = H // 2
NORM = 100.0

NW = 32
EW = E // NW

CW = 80
RING = 5
NSL = 5
ES = E // NSL
NCH = ES // (NW * CW)

SCS = 2000
SNCH = ES // (NW * SCS)
NP = 10240


def _pack_bf16(lo_f32, hi_f32):
    ulo = lax.bitcast_convert_type(lo_f32, jnp.int32)
    uhi = lax.bitcast_convert_type(hi_f32, jnp.int32)
    lo = lax.shift_right_logical(ulo + 0x8000, 16)
    hi = (uhi + 0x8000) & jnp.int32(-65536)
    return lo | hi


def _unpack_bf16(packed_i32):
    lo = lax.bitcast_convert_type(lax.shift_left(packed_i32, 16), jnp.float32)
    hi = lax.bitcast_convert_type(packed_i32 & jnp.int32(-65536), jnp.float32)
    return lo, hi


def _pre_body(h_ref, w1ae_ref, w1ao_ref, w1be_ref, w1bo_ref, t_ref):
    hblk = h_ref[...]
    pae = jnp.dot(hblk, w1ae_ref[...], preferred_element_type=jnp.float32)
    pao = jnp.dot(hblk, w1ao_ref[...], preferred_element_type=jnp.float32)
    pbe = jnp.dot(hblk, w1be_ref[...], preferred_element_type=jnp.float32)
    pbo = jnp.dot(hblk, w1bo_ref[...], preferred_element_type=jnp.float32)
    t_ref[...] = jnp.concatenate(
        [_pack_bf16(pae, pao), _pack_bf16(pbe, pbo)], axis=1)


def _precompute(h, w1ae, w1ao, w1be, w1bo):
    bn = 2000
    grid = (N // bn,)
    wspec = pl.BlockSpec((H, HH), lambda i: (0, 0))
    return pl.pallas_call(
        _pre_body,
        grid=grid,
        in_specs=[pl.BlockSpec((bn, H), lambda i: (i, 0))] + [wspec] * 4,
        out_specs=pl.BlockSpec((bn, H), lambda i: (i, 0)),
        out_shape=jax.ShapeDtypeStruct((N, H), jnp.int32),
    )(h, w1ae, w1ao, w1be, w1bo)


def _gather_body(t_hbm, row_hbm, col_hbm, ga_hbm, gb_hbm,
                 idxa, idxb, rawa, rawb, bufa, bufb, *sems):
    gsa = sems[0:RING]
    gsb = sems[RING:2 * RING]
    wsa = sems[2 * RING:3 * RING]
    wsb = sems[3 * RING:4 * RING]
    c = lax.axis_index("c")
    s = lax.axis_index("s")
    wid = s * 2 + c
    cbase = wid * NCH
    iota = lax.iota(jnp.int32, 16)
    EWH = NCH * CW // 2

    pltpu.sync_copy(row_hbm.at[pl.ds(wid * EWH, EWH)], rawa.at[pl.ds(0, EWH)])
    pltpu.sync_copy(row_hbm.at[pl.ds(ES // 2 + wid * EWH, EWH)],
                    rawa.at[pl.ds(EWH, EWH)])
    pltpu.sync_copy(col_hbm.at[pl.ds(wid * EWH, EWH)], rawb.at[pl.ds(0, EWH)])
    pltpu.sync_copy(col_hbm.at[pl.ds(ES // 2 + wid * EWH, EWH)],
                    rawb.at[pl.ds(EWH, EWH)])

    def ilv(g, carry):
        l16 = g * 16 + iota
        msk = l16 < EWH
        for raw, idx in ((rawa, idxa), (rawb, idxb)):
            lo16 = raw[pl.ds(g * 16, 16)]
            hi16 = raw[pl.ds(EWH + g * 16, 16)]
            plsc.store_scatter(idx, [2 * l16], lo16, mask=msk)
            plsc.store_scatter(idx, [2 * l16 + 1], hi16, mask=msk)
        return carry

    lax.fori_loop(0, (EWH + 15) // 16, ilv, 0)

    def do_fire(j, b):
        off = j * CW
        pltpu.async_copy(t_hbm.at[idxa.at[pl.ds(off, CW)]], bufa.at[b], gsa[b])
        pltpu.async_copy(t_hbm.at[idxb.at[pl.ds(off, CW)]], bufb.at[b], gsb[b])

    def do_writeout(j, b):
        off = j * CW
        pltpu.make_async_copy(t_hbm.at[idxa.at[pl.ds(off, CW)]],
                              bufa.at[b], gsa[b]).wait()
        pltpu.make_async_copy(t_hbm.at[idxb.at[pl.ds(off, CW)]],
                              bufb.at[b], gsb[b]).wait()
        off = (cbase + j) * CW
        pltpu.async_copy(bufa.at[b], ga_hbm.at[pl.ds(off, CW)], wsa[b])
        pltpu.async_copy(bufb.at[b], gb_hbm.at[pl.ds(off, CW)], wsb[b])

    def drain_writeout(b):
        pltpu.make_async_copy(bufa.at[b], ga_hbm.at[pl.ds(0, CW)], wsa[b]).wait()
        pltpu.make_async_copy(bufb.at[b], gb_hbm.at[pl.ds(0, CW)], wsb[b]).wait()

    def outer(go, carry):
        for b in range(RING):
            j = go * RING + b
            @pl.when(j >= 2)
            def _():
                do_writeout(j - 2, (b - 2) % RING)

            @pl.when(go >= 1)
            def _():
                drain_writeout(b)

            do_fire(j, b)
        return carry

    lax.fori_loop(0, NCH // RING, outer, 0)
    for j in (NCH - 2, NCH - 1):
        do_writeout(j, j % RING)
    for b in range(RING):
        drain_writeout(b)


def _gather(t2, row2, col2):
    mesh = plsc.VectorSubcoreMesh(core_axis_name="c", subcore_axis_name="s")
    k = pl.kernel(
        _gather_body,
        out_type=(
            jax.ShapeDtypeStruct((ES, HH), jnp.int32),
            jax.ShapeDtypeStruct((ES, HH), jnp.int32),
        ),
        mesh=mesh,
        scratch_types=[
            pltpu.VMEM((NCH * CW,), jnp.int32),
            pltpu.VMEM((NCH * CW,), jnp.int32),
            pltpu.VMEM((NCH * CW + 16,), jnp.int32),
            pltpu.VMEM((NCH * CW + 16,), jnp.int32),
            pltpu.VMEM((RING, CW, HH), jnp.int32),
            pltpu.VMEM((RING, CW, HH), jnp.int32),
        ] + [pltpu.SemaphoreType.DMA] * (4 * RING),
        compiler_params=pltpu.CompilerParams(use_tc_tiling_on_sc=False,
                                             needs_layout_passes=False),
    )
    return k(t2, row2, col2)


BK = 6400
BKH = BK // 2
BKR = BKH // H


def _mlp_body(ga_ref, gb_ref, eaL_ref, eaR_ref, w1ce_ref, w1co_ref,
              b1e_ref, b1o_ref, w2te_ref, w2to_ref, b2_ref, w3_ref,
              phiL_ref, phiR_ref):
    ale, alo = _unpack_bf16(ga_ref[...])
    ble, blo = _unpack_bf16(gb_ref[...])

    def half(lane0, ea_ref, phi_ref):
        sl = (slice(None), slice(lane0, lane0 + HH))
        xe = ale[sl] + ble[sl] + b1e_ref[...]
        xo = alo[sl] + blo[sl] + b1o_ref[...]
        ea3 = ea_ref[...][0][:, :, None]
        xe = xe + jnp.reshape(
            ea3 * jnp.reshape(w1ce_ref[...], (1, 1, HH)), (BKH, HH))
        xo = xo + jnp.reshape(
            ea3 * jnp.reshape(w1co_ref[...], (1, 1, HH)), (BKH, HH))
        xe = xe * (1.0 / (1.0 + jnp.exp(-xe)))
        xo = xo * (1.0 / (1.0 + jnp.exp(-xo)))
        x = (jnp.dot(xe.astype(jnp.bfloat16), w2te_ref[...],
                     preferred_element_type=jnp.float32)
             + jnp.dot(xo.astype(jnp.bfloat16), w2to_ref[...],
                       preferred_element_type=jnp.float32)
             + b2_ref[...])
        x = x * (1.0 / (1.0 + jnp.exp(-x)))
        phi = jnp.sum(jnp.reshape(x * w3_ref[...], (BKR, H, H)), axis=2)
        phi_ref[...] = jnp.reshape(phi, (1, BKR, H))

    half(0, eaL_ref, phiL_ref)
    half(HH, eaR_ref, phiR_ref)


def _edge_mlp(ga2, gb2, eaL, eaR, w1ce, w1co, b1e, b1o, w2te, w2to, b2r, w3r):
    grid = (ES // BK,)
    hspec = pl.BlockSpec((1, HH), lambda i: (0, 0))
    wspec = pl.BlockSpec((1, H), lambda i: (0, 0))
    easpec = pl.BlockSpec((1, BKR, H), lambda i: (i, 0, 0))
    pspec = pl.BlockSpec((1, BKR, H), lambda i: (i, 0, 0))
    pshape = jax.ShapeDtypeStruct((ES // BK, BKR, H), jnp.float32)
    return pl.pallas_call(
        _mlp_body,
        grid=grid,
        in_specs=[
            pl.BlockSpec((BKH, H), lambda i: (i, 0)),
            pl.BlockSpec((BKH, H), lambda i: (i, 0)),
            easpec,
            easpec,
            hspec,
            hspec,
            hspec,
            hspec,
            pl.BlockSpec((HH, H), lambda i: (0, 0)),
            pl.BlockSpec((HH, H), lambda i: (0, 0)),
            wspec,
            wspec,
        ],
        out_specs=[pspec, pspec],
        out_shape=[pshape, pshape],
    )(ga2, gb2, eaL.reshape(ES // BK, BKR, H), eaR.reshape(ES // BK, BKR, H),
      w1ce, w1co, b1e, b1o, w2te, w2to, b2r, w3r)


def _scatter_body(row_hbm, phi_hbm, cd0_hbm, cd1_hbm, cd2_hbm, part_hbm,
                  idx_v, phi_v, cd0_v, cd1_v, cd2_v, acc_v):
    c = lax.axis_index("c")
    s = lax.axis_index("s")
    wid = s * 2 + c
    base = wid * (ES // NW)
    iota = lax.iota(jnp.int32, 16)
    zeros16 = jnp.zeros((16,), jnp.float32)

    def zero(i, carry):
        for k in range(10):
            acc_v[pl.ds(i * 160 + k * 16, 16)] = zeros16
        return carry

    lax.fori_loop(0, 3 * NP // 160, zero, 0)

    def chunk(j, carry):
        off = base + j * SCS
        pltpu.sync_copy(row_hbm.at[pl.ds(off, SCS)], idx_v)
        pltpu.sync_copy(phi_hbm.at[pl.ds(off, SCS)], phi_v)
        pltpu.sync_copy(cd0_hbm.at[pl.ds(off, SCS)], cd0_v)
        pltpu.sync_copy(cd1_hbm.at[pl.ds(off, SCS)], cd1_v)
        pltpu.sync_copy(cd2_hbm.at[pl.ds(off, SCS)], cd2_v)

        def grp(g, carry2):
            p16 = phi_v[pl.ds(g * 16, 16)]
            e16 = idx_v[pl.ds(g * 16, 16)]
            for comp, cdv in enumerate((cd0_v, cd1_v, cd2_v)):
                cdc = cdv[pl.ds(g * 16, 16)]
                plsc.addupdate_scatter(acc_v, [e16 + comp * NP], p16 * cdc)
            return carry2

        lax.fori_loop(0, SCS // 16, grp, 0)
        return carry

    lax.fori_loop(0, SNCH, chunk, 0)
    pltpu.sync_copy(acc_v, part_hbm.at[wid])


def _scatter(row, phi, cd0, cd1, cd2):
    mesh = plsc.VectorSubcoreMesh(core_axis_name="c", subcore_axis_name="s")
    k = pl.kernel(
        _scatter_body,
        out_type=jax.ShapeDtypeStruct((NW, 3 * NP), jnp.float32),
        mesh=mesh,
        scratch_types=[
            pltpu.VMEM((SCS,), jnp.int32),
            pltpu.VMEM((SCS,), jnp.float32),
            pltpu.VMEM((SCS,), jnp.float32),
            pltpu.VMEM((SCS,), jnp.float32),
            pltpu.VMEM((SCS,), jnp.float32),
            pltpu.VMEM((3 * NP,), jnp.float32),
        ],
        compiler_params=pltpu.CompilerParams(needs_layout_passes=False),
    )
    return k(row, phi, cd0, cd1, cd2)


def _comb_body(*refs):
    part_refs = refs[:NSL]
    coordt_ref, out_ref = refs[NSL], refs[NSL + 1]
    p = part_refs[0][...]
    for pr in part_refs[1:]:
        p = p + pr[...]
    planes = [jnp.sum(p[:, comp * NP:(comp + 1) * NP], axis=0)[:N]
              for comp in range(3)]
    out_ref[...] = coordt_ref[...] + jnp.stack(planes, axis=0) * (1.0 / NORM)


def _combine(parts_list, coordt):
    return pl.pallas_call(
        _comb_body,
        grid=(1,),
        in_specs=[pl.BlockSpec((NW, 3 * NP), lambda i: (0, 0))] * NSL + [
            pl.BlockSpec((3, N), lambda i: (0, 0)),
        ],
        out_specs=pl.BlockSpec((3, N), lambda i: (0, 0)),
        out_shape=jax.ShapeDtypeStruct((3, N), jnp.float32),
    )(*parts_list, coordt)


def kernel(h, coord, edge_index, coord_diff, edge_attr, W1, b1, W2, b2, W3):
    row = edge_index[0]
    col = edge_index[1]
    w1at = W1[:, :H].T
    w1bt = W1[:, H:2 * H].T
    w1c = W1[:, 2 * H:].T
    w2t = W2.T
    w1ce, w1co = w1c[:, :HH], w1c[:, HH:]
    b1e = b1[:HH].reshape(1, HH)
    b1o = b1[HH:].reshape(1, HH)
    w2te = w2t[:HH, :].astype(jnp.bfloat16)
    w2to = w2t[HH:, :].astype(jnp.bfloat16)
    b2r = b2.reshape(1, H)
    w3r = W3
    t2 = _precompute(h, w1at[:, :HH], w1at[:, HH:],
                     w1bt[:, :HH], w1bt[:, HH:]).reshape(2 * N, HH)
    row2 = row * 2
    col2 = col * 2 + 1
    eat = edge_attr.T
    cdt = coord_diff.T
    parts_list = []
    for s in range(NSL):
        ga, gb = _gather(t2, row2[s * ES:(s + 1) * ES],
                         col2[s * ES:(s + 1) * ES])
        phiL, phiR = _edge_mlp(ga.reshape(ES // 2, H), gb.reshape(ES // 2, H),
                               eat[:, s * ES:s * ES + ES // 2],
                               eat[:, s * ES + ES // 2:(s + 1) * ES],
                               w1ce, w1co, b1e, b1o, w2te, w2to, b2r, w3r)
        phi_s = jnp.concatenate(
            [phiL.reshape(ES // 2), phiR.reshape(ES // 2)], axis=0)
        parts_list.append(_scatter(
            row[s * ES:(s + 1) * ES], phi_s,
            cdt[0, s * ES:(s + 1) * ES],
            cdt[1, s * ES:(s + 1) * ES],
            cdt[2, s * ES:(s + 1) * ES]))
    return _combine(parts_list, coord.T).T
